# baseline (speedup 1.0000x reference)
import functools

import jax
import jax.numpy as jnp
from jax.experimental import pallas as pl
from jax.experimental.pallas import tpu as pltpu

_MIN = jnp.finfo(jnp.float32).min


def _gate_block(h, w, n_per, n_experts):
    logits = jnp.dot(h.astype(jnp.bfloat16), w,
                     preferred_element_type=jnp.float32)
    tm, width = logits.shape
    cols = jax.lax.broadcasted_iota(jnp.int32, (tm, width), 1)
    cmasked = jnp.where(cols >= n_experts, logits, -jnp.inf)
    cmax = jnp.max(cmasked, axis=1, keepdims=True)
    ci = jnp.min(jnp.where(cmasked == cmax, cols, width), axis=1) - n_experts
    ecols = jax.lax.broadcasted_iota(jnp.int32, (tm, n_experts), 1)
    keep = (ecols // n_per) == ci[:, None]
    return jnp.where(keep, logits[:, :n_experts], _MIN)


def _gate_kernel(ha_ref, hb_ref, w_ref, out_ref, *, n_per: int,
                 n_experts: int):
    w = w_ref[...]
    out_ref[0] = _gate_block(ha_ref[...], w, n_per, n_experts)
    out_ref[1] = _gate_block(hb_ref[...], w, n_per, n_experts)


@jax.jit
def kernel(hidden_states, Wc, We):
    B, S, D = hidden_states.shape
    num_clusters = Wc.shape[0]
    n_per = We.shape[1]
    n_experts = num_clusters * n_per
    T = B * S
    h = hidden_states.reshape(T, D)
    w_all = jnp.concatenate([We.reshape(n_experts, D), Wc],
                            axis=0).T.astype(jnp.bfloat16)

    TM = 512
    while (T // 2) % TM:
        TM //= 2
    T2 = T // 2
    grid = T2 // TM

    out = pl.pallas_call(
        functools.partial(_gate_kernel, n_per=n_per, n_experts=n_experts),
        grid=(grid,),
        in_specs=[
            pl.BlockSpec((TM, D), lambda i: (i, 0)),
            pl.BlockSpec((TM, D), lambda i, g=grid: (i + g, 0)),
            pl.BlockSpec((D, n_experts + num_clusters), lambda i: (0, 0)),
        ],
        out_specs=pl.BlockSpec((2, TM, n_experts), lambda i: (0, i, 0)),
        out_shape=jax.ShapeDtypeStruct((2, T2, n_experts), jnp.float32),
        compiler_params=pltpu.CompilerParams(
            dimension_semantics=("parallel",),
        ),
    )(h, h, w_all)
    return out.reshape(B, S, n_experts)
